# baseline (speedup 1.0000x reference)
import jax
import jax.numpy as jnp
from jax.experimental import pallas as pl

B, N, D, H = 64, 1024, 192, 48
BT = 4096
G = (B * N) // BT


def _body(x_ref, lu_ref, mt_ref, w1_ref, b1_ref, w2_ref, b2_ref,
          out_ref, m_ref):
    xb = x_ref[...]
    ht = jax.lax.dot_general(
        w1_ref[...], xb, (((1,), (1,)), ((), ())),
        preferred_element_type=jnp.float32)
    g = jax.nn.gelu(ht + b1_ref[...])
    logit = jax.lax.dot_general(
        w2_ref[...], g, (((1,), (0,)), ((), ())),
        preferred_element_type=jnp.float32) + b2_ref[...]
    s = logit - lu_ref[0]
    m_ref[...] = (s > 0)[None].astype(jnp.int8)
    ones_row = jnp.ones((1, D), jnp.float32)
    smat = jax.lax.dot_general(
        s, ones_row, (((0,), (0,)), ((), ())),
        preferred_element_type=jnp.float32)
    out_ref[...] = jnp.where(smat > 0, mt_ref[...], xb)


def kernel(x, mask_token, W1, b1, W2, b2):
    u = jax.random.uniform(jax.random.key(42), (B, N, 1), jnp.float32)
    lu = (jnp.log(u) - jnp.log1p(-u)).reshape(G, 1, BT)
    x2 = x.reshape(B * N, D)
    masked, m8 = pl.pallas_call(
        _body,
        grid=(G,),
        in_specs=[
            pl.BlockSpec((BT, D), lambda g: (g, 0)),
            pl.BlockSpec((1, 1, BT), lambda g: (g, 0, 0)),
            pl.BlockSpec((1, D), lambda g: (0, 0)),
            pl.BlockSpec((H, D), lambda g: (0, 0)),
            pl.BlockSpec((H, 1), lambda g: (0, 0)),
            pl.BlockSpec((1, H), lambda g: (0, 0)),
            pl.BlockSpec((1, 1), lambda g: (0, 0)),
        ],
        out_specs=[
            pl.BlockSpec((BT, D), lambda g: (g, 0)),
            pl.BlockSpec((1, 1, BT), lambda g: (g, 0, 0)),
        ],
        out_shape=[
            jax.ShapeDtypeStruct((B * N, D), jnp.float32),
            jax.ShapeDtypeStruct((G, 1, BT), jnp.int8),
        ],
    )(x2, lu, mask_token.reshape(1, D), W1, b1.reshape(H, 1),
      W2, b2.reshape(1, 1))
    return masked.reshape(B, N, D), m8.reshape(B, N).astype(jnp.bool_)
